# baseline (speedup 1.0000x reference)
import functools

import jax
import jax.numpy as jnp
from jax import lax
from jax.experimental import pallas as pl
from jax.experimental.pallas import tpu as pltpu
from jax.experimental.pallas import tpu_sc as plsc

_CA = 160
_B = 1024
_N = 50
_NH = 25
_CW = 40
_NG = _B // 16


def _make_sc_call(layers):
    nl = len(layers)
    wpl = 32 // nl
    assert wpl == 8, "this mapping assumes 8 workers per layer"
    mesh = plsc.VectorSubcoreMesh(core_axis_name="c", subcore_axis_name="s")
    shp = jax.ShapeDtypeStruct((_N, _CA, _B), jnp.float32)

    def body(idx_hbm, diag_hbm, zeros_hbm, *refs):
        outs = refs[:nl]
        idx_v, diag_v, buf0, buf1, sem0, sem1 = refs[nl:]
        bufs = (buf0, buf1)
        sems = (sem0, sem1)
        w = lax.axis_index("s") * 2 + lax.axis_index("c")
        iota = lax.iota(jnp.int32, 16)
        zeros16 = jnp.zeros((16,), jnp.float32)
        z16 = jnp.zeros((16,), jnp.int32)

        for k in range(nl):
            li = layers[k]

            @pl.when(w // 8 == k)
            def _layer():
                r = w % 8
                n0 = (r // 4) * _NH
                c0 = (r % 4) * _CW
                pltpu.sync_copy(
                    idx_hbm.at[pl.ds((li * _N + n0) * _B, _NH * _B)], idx_v)
                pltpu.sync_copy(diag_hbm, diag_v)
                pltpu.sync_copy(zeros_hbm, buf0)
                pltpu.sync_copy(zeros_hbm, buf1)

                def dma(ci, buf, sem):
                    dst = outs[k].at[pl.ds(n0 + ci, 1), pl.ds(c0, _CW), :]
                    return pltpu.make_async_copy(buf, dst, sem)

                def fill(ci, buf, restore_ci):
                    def grp(g, _):
                        bvec = iota + g * 16
                        if restore_ci is not None:
                            ov = idx_v[pl.ds(restore_ci * _B + g * 16, 16)]
                            ocv = ov - c0
                            om = (ocv >= 0) & (ocv < _CW)
                            plsc.store_scatter(buf, [z16, ocv, bvec],
                                               zeros16, mask=om)
                        idxvec = idx_v[pl.ds(ci * _B + g * 16, 16)]
                        cv = idxvec - c0
                        m = (cv >= 0) & (cv < _CW)
                        vals = plsc.load_gather(diag_v, [idxvec], mask=m)
                        plsc.store_scatter(buf, [z16, cv, bvec], vals, mask=m)
                        return 0
                    lax.fori_loop(0, _NG, grp, 0)

                for b in range(2):
                    fill(b, bufs[b], None)
                    dma(b, bufs[b], sems[b]).start()

                def pair(gg, _):
                    for b in range(2):
                        ci = gg * 2 + b
                        dma(ci - 2, bufs[b], sems[b]).wait()
                        fill(ci, bufs[b], ci - 2)
                        dma(ci, bufs[b], sems[b]).start()
                    return 0
                lax.fori_loop(1, 12, pair, 0)

                dma(22, bufs[0], sems[0]).wait()
                fill(24, bufs[0], 22)
                dma(24, bufs[0], sems[0]).start()
                dma(23, bufs[1], sems[1]).wait()
                dma(24, bufs[0], sems[0]).wait()

    return functools.partial(
        pl.kernel,
        mesh=mesh,
        compiler_params=pltpu.CompilerParams(needs_layout_passes=False),
        out_type=[shp] * nl,
        scratch_types=[
            pltpu.VMEM((_NH * _B,), jnp.int32),
            pltpu.VMEM((_CA,), jnp.float32),
            pltpu.VMEM((1, _CW, _B), jnp.float32),
            pltpu.VMEM((1, _CW, _B), jnp.float32),
            pltpu.SemaphoreType.DMA,
            pltpu.SemaphoreType.DMA,
        ])(body)


def kernel(nei_rel_list, one_hot):
    idx_t = jnp.swapaxes(nei_rel_list, 1, 2)
    idx_flat = idx_t.reshape(-1)
    diag = jnp.diagonal(one_hot)
    zeros = jnp.zeros((1, _CW, _B), jnp.float32)
    outs = _make_sc_call((0, 1, 2, 3))(idx_flat, diag, zeros)
    return tuple(jnp.transpose(t, (2, 0, 1)) for t in outs)
